# baseline (speedup 1.0000x reference)
import functools

import jax
import jax.numpy as jnp
from jax import lax
from jax.experimental import pallas as pl
from jax.experimental.pallas import tpu as pltpu
from jax.experimental.pallas import tpu_sc as plsc

B = 16384
D = 64
NC = 2
NS = 16
NW = NC * NS
BPW = B // NW
CH = 128
NCH = BPW // CH


def _body(hidx_hbm, ridx_hbm, tidx_hbm, ent_hbm, relt_hbm,
          hout, rout, tout,
          hidx, ridx, tidx, hrows, rrows, trows, gsem, wsem):
    wid = lax.axis_index("s") * NC + lax.axis_index("c")
    base = wid * BPW

    pltpu.sync_copy(hidx_hbm.at[wid], hidx)
    pltpu.sync_copy(ridx_hbm.at[wid], ridx)
    pltpu.sync_copy(tidx_hbm.at[wid], tidx)

    copies = []
    for j in range(NCH):
        sl = pl.ds(j * CH, CH)
        copies.append(pltpu.async_copy(ent_hbm.at[hidx.at[j]], hrows.at[sl], gsem))
        copies.append(pltpu.async_copy(ent_hbm.at[tidx.at[j]], trows.at[sl], gsem))
        copies.append(pltpu.async_copy(relt_hbm.at[ridx.at[j]], rrows.at[sl], gsem))
    for c in copies:
        c.wait()

    out = pl.ds(base, BPW)
    w = [pltpu.async_copy(hrows, hout.at[out], wsem),
         pltpu.async_copy(rrows, rout.at[out], wsem),
         pltpu.async_copy(trows, tout.at[out], wsem)]
    for c in w:
        c.wait()


_mesh = plsc.VectorSubcoreMesh(core_axis_name="c", subcore_axis_name="s")

_gather = functools.partial(
    pl.kernel,
    out_type=(
        jax.ShapeDtypeStruct((B, D), jnp.float32),
        jax.ShapeDtypeStruct((B, D), jnp.float32),
        jax.ShapeDtypeStruct((B, D), jnp.float32),
    ),
    mesh=_mesh,
    scratch_types=[
        pltpu.VMEM((NCH, CH), jnp.int32),
        pltpu.VMEM((NCH, CH), jnp.int32),
        pltpu.VMEM((NCH, CH), jnp.int32),
        pltpu.VMEM((BPW, D), jnp.float32),
        pltpu.VMEM((BPW, D), jnp.float32),
        pltpu.VMEM((BPW, D), jnp.float32),
        pltpu.SemaphoreType.DMA,
        pltpu.SemaphoreType.DMA,
    ],
    compiler_params=pltpu.CompilerParams(use_tc_tiling_on_sc=False),
)(_body)


@jax.jit
def kernel(head, relation, tail, entity_table, relation_table):
    h = head.astype(jnp.int32).reshape(NW, NCH, CH)
    r = relation.astype(jnp.int32).reshape(NW, NCH, CH)
    t = tail.astype(jnp.int32).reshape(NW, NCH, CH)
    head_emb, relation_emb, tail_emb = _gather(h, r, t, entity_table,
                                               relation_table)
    return (head_emb, relation_emb, tail_emb)
